# baseline (speedup 1.0000x reference)
import jax
import jax.numpy as jnp
from jax.experimental import pallas as pl
from jax.experimental.pallas import tpu as pltpu

B, D, H = 4096, 1024, 2048
BM = 1024
NH = 4
KH = H // NH


def _dense_body(x_ref, w1_ref, b1_ref, w2_ref, b2_ref, p_ref, o_ref, pick_scr):
    n = pl.program_id(0)
    b = pl.program_id(1)
    xb = x_ref[...]

    @pl.when(n == 0)
    def _():
        p = p_ref[...]
        diff0 = xb - p[0:1, :]
        diff1 = xb - p[1:2, :]
        d0 = jnp.sqrt(jnp.sum(diff0 * diff0, axis=1, keepdims=True))
        d1 = jnp.sqrt(jnp.sum(diff1 * diff1, axis=1, keepdims=True))
        pick_scr[pl.ds(b * BM, BM), :] = (d1 < d0).astype(jnp.float32)

    pick = pick_scr[pl.ds(b * BM, BM), :] > 0

    dn = (((1,), (1,)), ((), ()))
    h0 = jax.nn.relu(
        jax.lax.dot_general(xb, w1_ref[0], dn, preferred_element_type=jnp.float32)
        + b1_ref[0])
    o0 = jax.lax.dot_general(h0, w2_ref[0], dn, preferred_element_type=jnp.float32)
    h1 = jax.nn.relu(
        jax.lax.dot_general(xb, w1_ref[1], dn, preferred_element_type=jnp.float32)
        + b1_ref[1])
    o1 = jax.lax.dot_general(h1, w2_ref[1], dn, preferred_element_type=jnp.float32)
    osel = jnp.where(pick, o1, o0)

    @pl.when(n == 0)
    def _():
        b2sel = jnp.where(pick, b2_ref[1], b2_ref[0])
        o_ref[...] = osel + b2sel

    @pl.when(n > 0)
    def _():
        o_ref[...] += osel


def kernel(x, w1, b1, w2, b2, prototypes):
    b1r = b1.reshape(2, 1, H)
    b2r = b2.reshape(2, 1, 2)
    out = pl.pallas_call(
        _dense_body,
        grid=(NH, B // BM),
        in_specs=[
            pl.BlockSpec((BM, D), lambda n, b: (b, 0)),
            pl.BlockSpec((2, KH, D), lambda n, b: (0, n, 0)),
            pl.BlockSpec((2, 1, KH), lambda n, b: (0, 0, n)),
            pl.BlockSpec((2, 2, KH), lambda n, b: (0, 0, n)),
            pl.BlockSpec((2, 1, 2), lambda n, b: (0, 0, 0)),
            pl.BlockSpec((2, D), lambda n, b: (0, 0)),
        ],
        out_specs=pl.BlockSpec((BM, 2), lambda n, b: (b, 0)),
        out_shape=jax.ShapeDtypeStruct((B, 2), jnp.float32),
        scratch_shapes=[pltpu.VMEM((B, 1), jnp.float32)],
    )(x, w1, b1r, w2, b2r, prototypes)
    return out


# speedup vs baseline: 1.1627x; 1.1627x over previous
import jax
import jax.numpy as jnp
from jax.experimental import pallas as pl

B, D, H = 4096, 1024, 2048
BM = 1024


def _dense_body(x_ref, w1_ref, b1_ref, w2_ref, b2_ref, p_ref, o_ref):
    xb = x_ref[...]
    p = p_ref[...]
    diff0 = xb - p[0:1, :]
    diff1 = xb - p[1:2, :]
    d0 = jnp.sqrt(jnp.sum(diff0 * diff0, axis=1, keepdims=True))
    d1 = jnp.sqrt(jnp.sum(diff1 * diff1, axis=1, keepdims=True))
    pick1 = d1 < d0

    dn = (((1,), (1,)), ((), ()))
    h0 = jax.nn.relu(
        jax.lax.dot_general(xb, w1_ref[0], dn, preferred_element_type=jnp.float32)
        + b1_ref[0])
    o0 = (jax.lax.dot_general(h0, w2_ref[0], dn, preferred_element_type=jnp.float32)
          + b2_ref[0])
    h1 = jax.nn.relu(
        jax.lax.dot_general(xb, w1_ref[1], dn, preferred_element_type=jnp.float32)
        + b1_ref[1])
    o1 = (jax.lax.dot_general(h1, w2_ref[1], dn, preferred_element_type=jnp.float32)
          + b2_ref[1])
    o_ref[...] = jnp.where(pick1, o1, o0)


def kernel(x, w1, b1, w2, b2, prototypes):
    b1r = b1.reshape(2, 1, H)
    b2r = b2.reshape(2, 1, 2)
    out = pl.pallas_call(
        _dense_body,
        grid=(B // BM,),
        in_specs=[
            pl.BlockSpec((BM, D), lambda i: (i, 0)),
            pl.BlockSpec((2, H, D), lambda i: (0, 0, 0)),
            pl.BlockSpec((2, 1, H), lambda i: (0, 0, 0)),
            pl.BlockSpec((2, 2, H), lambda i: (0, 0, 0)),
            pl.BlockSpec((2, 1, 2), lambda i: (0, 0, 0)),
            pl.BlockSpec((2, D), lambda i: (0, 0)),
        ],
        out_specs=pl.BlockSpec((BM, 2), lambda i: (i, 0)),
        out_shape=jax.ShapeDtypeStruct((B, 2), jnp.float32),
    )(x, w1, b1r, w2, b2r, prototypes)
    return out
